# baseline (speedup 1.0000x reference)
import functools

import jax
import jax.numpy as jnp
from jax import lax
from jax.experimental import pallas as pl
from jax.experimental.pallas import tpu as pltpu
from jax.experimental.pallas import tpu_sc as plsc

_K = 64
_ROWS = 128
_COLS = 32768
_ROWS_PER_W = _ROWS // 32


def _rank_descend(ts, gs, k):
    iota = lax.iota(jnp.int32, 16)
    sg = [None] * 17
    sg[16] = jnp.int32(0)
    for i in range(15, -1, -1):
        sg[i] = sg[i + 1] + gs[i]
    G = jnp.int32(0)
    for i in range(16):
        G = jnp.where(sg[i] >= k, jnp.int32(i), G)
    sgn = jnp.int32(0)
    v = ts[0]
    for i in range(16):
        is_g = G == jnp.int32(i)
        sgn = jnp.where(is_g, sg[i + 1], sgn)
        v = jnp.where(is_g, ts[i], v)
    kk = k - sgn
    s = lax.rev(plsc.cumsum(lax.rev(v, (0,))), (0,))
    m = s >= kk
    bl = jnp.max(jnp.where(m, iota, jnp.int32(-1)))
    hb = jnp.max(jnp.where(iota == bl, v, jnp.int32(0)))
    s_at = jnp.max(jnp.where(iota == bl, s, jnp.int32(0)))
    return G * 16 + bl, kk - (s_at - hb)


def _scan_shared(hist_ref, k):
    zeros = jnp.zeros(16, jnp.int32)
    ts, gs = [], []
    for i in range(16):
        t = hist_ref[pl.ds(i * 16, 16)]
        ts.append(t)
        gs.append(jnp.sum(t))
        hist_ref[pl.ds(i * 16, 16)] = zeros
    return _rank_descend(ts, gs, k)


def _scan_private(histp_ref, k):
    zeros = jnp.zeros(16, jnp.int32)
    ts, gs = [], []
    for g in range(16):
        acc = histp_ref[pl.ds(g * 16, 16)]
        for l in range(1, 16):
            acc = acc + histp_ref[pl.ds(l * 256 + g * 16, 16)]
        ts.append(acc)
        gs.append(jnp.sum(acc))

    @plsc.parallel_loop(0, 4096, 16, unroll=8)
    def _zero(o):
        histp_ref[pl.ds(o, 16)] = zeros

    return _rank_descend(ts, gs, k)


_mesh = plsc.VectorSubcoreMesh(core_axis_name="c", subcore_axis_name="s")


@functools.partial(
    pl.kernel,
    out_type=jax.ShapeDtypeStruct((_ROWS, _COLS), jnp.int32),
    mesh=_mesh,
    scratch_types=[
        pltpu.VMEM((_COLS,), jnp.int32),
        pltpu.VMEM((_COLS,), jnp.int32),
        pltpu.VMEM((4096,), jnp.int32),
        pltpu.VMEM((256,), jnp.int32),
        pltpu.SemaphoreType.DMA,
        pltpu.SemaphoreType.DMA,
        pltpu.SemaphoreType.DMA,
        pltpu.SemaphoreType.DMA,
    ],
    compiler_params=pltpu.CompilerParams(needs_layout_passes=False),
)
def _sc_ksparse(x_hbm, out_hbm, key_a, key_b, histp_ref, hist_ref,
                sem_in0, sem_in1, sem_out0, sem_out1):
    wid = lax.axis_index("s") * 2 + lax.axis_index("c")
    iota = lax.iota(jnp.int32, 16)
    lane_base = iota * 256
    ones = jnp.ones(16, jnp.int32)
    zeros = jnp.zeros(16, jnp.int32)
    for i in range(16):
        hist_ref[pl.ds(i * 16, 16)] = zeros

    @plsc.parallel_loop(0, 4096, 16, unroll=8)
    def _z0(o):
        histp_ref[pl.ds(o, 16)] = zeros

    bufs = [key_a, key_b]
    sems_in = [sem_in0, sem_in1]
    sems_out = [sem_out0, sem_out1]
    base = wid * _ROWS_PER_W
    pltpu.make_async_copy(x_hbm.at[base], bufs[0], sems_in[0]).start()

    for jr in range(_ROWS_PER_W):
        r = base + jr
        key_v = bufs[jr % 2]
        pltpu.make_async_copy(x_hbm.at[r], key_v, sems_in[jr % 2]).wait()
        if jr + 1 < _ROWS_PER_W:
            nxt = bufs[(jr + 1) % 2]
            if jr >= 1:
                pltpu.make_async_copy(
                    nxt, out_hbm.at[r - 1], sems_out[(jr + 1) % 2]).wait()
            pltpu.make_async_copy(x_hbm.at[r + 1], nxt, sems_in[(jr + 1) % 2]).start()

        @plsc.parallel_loop(0, _COLS, 16, unroll=8)
        def p1(o):
            s = key_v[pl.ds(o, 16)]
            ik = s ^ lax.shift_right_logical(lax.shift_right_arithmetic(s, 31), 1)
            key_v[pl.ds(o, 16)] = ik
            b0 = lax.shift_right_arithmetic(ik, 24) + 128
            plsc.addupdate_scatter(histp_ref, [lane_base + b0], ones)

        B0, k1 = _scan_private(histp_ref, jnp.int32(_K))

        @plsc.parallel_loop(0, _COLS, 16, unroll=8)
        def p2(o):
            ik = key_v[pl.ds(o, 16)]
            m = (lax.shift_right_arithmetic(ik, 24) + 128) == B0
            b = jnp.bitwise_and(lax.shift_right_arithmetic(ik, 16), 255)
            plsc.addupdate_scatter(hist_ref, [b], ones, mask=m)

        B1, k2 = _scan_shared(hist_ref, k1)
        t16 = (B0 - 128) * 256 + B1

        @plsc.parallel_loop(0, _COLS, 16, unroll=8)
        def p3(o):
            ik = key_v[pl.ds(o, 16)]
            m = lax.shift_right_arithmetic(ik, 16) == t16
            b = jnp.bitwise_and(lax.shift_right_arithmetic(ik, 8), 255)
            plsc.addupdate_scatter(hist_ref, [b], ones, mask=m)

        B2, k3 = _scan_shared(hist_ref, k2)
        t8 = t16 * 256 + B2

        @plsc.parallel_loop(0, _COLS, 16, unroll=8)
        def p4(o):
            ik = key_v[pl.ds(o, 16)]
            m = lax.shift_right_arithmetic(ik, 8) == t8
            b = jnp.bitwise_and(ik, 255)
            plsc.addupdate_scatter(hist_ref, [b], ones, mask=m)

        B3, _ = _scan_shared(hist_ref, k3)
        thr = t8 * 256 + B3

        @plsc.parallel_loop(0, _COLS, 16, unroll=8)
        def p5(o):
            ik = key_v[pl.ds(o, 16)]
            v = ik ^ lax.shift_right_logical(lax.shift_right_arithmetic(ik, 31), 1)
            key_v[pl.ds(o, 16)] = jnp.where(ik >= thr, v, jnp.int32(0))

        pltpu.make_async_copy(key_v, out_hbm.at[r], sems_out[jr % 2]).start()

    last = _ROWS_PER_W - 1
    pltpu.make_async_copy(
        bufs[(last - 1) % 2], out_hbm.at[base + last - 1],
        sems_out[(last - 1) % 2]).wait()
    pltpu.make_async_copy(
        bufs[last % 2], out_hbm.at[base + last], sems_out[last % 2]).wait()


def kernel(inputs):
    bits = lax.bitcast_convert_type(inputs, jnp.int32)
    out = _sc_ksparse(bits)
    return lax.bitcast_convert_type(out, jnp.float32)


# speedup vs baseline: 1.0001x; 1.0001x over previous
import functools

import jax
import jax.numpy as jnp
from jax import lax
from jax.experimental import pallas as pl
from jax.experimental.pallas import tpu as pltpu
from jax.experimental.pallas import tpu_sc as plsc

_K = 64
_ROWS = 128
_COLS = 32768
_ROWS_PER_W = _ROWS // 32


def _rank_descend(ts, gs, k):
    iota = lax.iota(jnp.int32, 16)
    sg = [None] * 17
    sg[16] = jnp.int32(0)
    for i in range(15, -1, -1):
        sg[i] = sg[i + 1] + gs[i]
    G = jnp.int32(0)
    for i in range(16):
        G = jnp.where(sg[i] >= k, jnp.int32(i), G)
    sgn = jnp.int32(0)
    v = ts[0]
    for i in range(16):
        is_g = G == jnp.int32(i)
        sgn = jnp.where(is_g, sg[i + 1], sgn)
        v = jnp.where(is_g, ts[i], v)
    kk = k - sgn
    s = lax.rev(plsc.cumsum(lax.rev(v, (0,))), (0,))
    m = s >= kk
    bl = jnp.max(jnp.where(m, iota, jnp.int32(-1)))
    hb = jnp.max(jnp.where(iota == bl, v, jnp.int32(0)))
    s_at = jnp.max(jnp.where(iota == bl, s, jnp.int32(0)))
    return G * 16 + bl, kk - (s_at - hb)


def _scan_shared(hist_ref, k):
    zeros = jnp.zeros(16, jnp.int32)
    ts, gs = [], []
    for i in range(16):
        t = hist_ref[pl.ds(i * 16, 16)]
        ts.append(t)
        gs.append(jnp.sum(t))
        hist_ref[pl.ds(i * 16, 16)] = zeros
    return _rank_descend(ts, gs, k)


def _scan_private(histp_ref, k):
    zeros = jnp.zeros(16, jnp.int32)
    ts, gs = [], []
    for g in range(16):
        acc = histp_ref[pl.ds(g * 16, 16)]
        for l in range(1, 16):
            acc = acc + histp_ref[pl.ds(l * 256 + g * 16, 16)]
        ts.append(acc)
        gs.append(jnp.sum(acc))

    @plsc.parallel_loop(0, 4096, 16, unroll=8)
    def _zero(o):
        histp_ref[pl.ds(o, 16)] = zeros

    return _rank_descend(ts, gs, k)


_mesh = plsc.VectorSubcoreMesh(core_axis_name="c", subcore_axis_name="s")


@functools.partial(
    pl.kernel,
    out_type=jax.ShapeDtypeStruct((_ROWS, _COLS), jnp.int32),
    mesh=_mesh,
    scratch_types=[
        pltpu.VMEM((_COLS,), jnp.int32),
        pltpu.VMEM((_COLS,), jnp.int32),
        pltpu.VMEM((_COLS,), jnp.int32),
        pltpu.VMEM((4096,), jnp.int32),
        pltpu.VMEM((256,), jnp.int32),
        pltpu.SemaphoreType.DMA,
        pltpu.SemaphoreType.DMA,
        pltpu.SemaphoreType.DMA,
        pltpu.SemaphoreType.DMA,
    ],
    compiler_params=pltpu.CompilerParams(needs_layout_passes=False),
)
def _sc_ksparse(x_hbm, out_hbm, in_a, in_b, key_v, histp_ref, hist_ref,
                sem_in0, sem_in1, sem_out0, sem_out1):
    wid = lax.axis_index("s") * 2 + lax.axis_index("c")
    iota = lax.iota(jnp.int32, 16)
    lane_base = iota * 256
    ones = jnp.ones(16, jnp.int32)
    zeros = jnp.zeros(16, jnp.int32)
    for i in range(16):
        hist_ref[pl.ds(i * 16, 16)] = zeros

    @plsc.parallel_loop(0, 4096, 16, unroll=8)
    def _z0(o):
        histp_ref[pl.ds(o, 16)] = zeros

    bufs = [in_a, in_b]
    sems_in = [sem_in0, sem_in1]
    sems_out = [sem_out0, sem_out1]
    base = wid * _ROWS_PER_W
    pltpu.make_async_copy(x_hbm.at[base], bufs[0], sems_in[0]).start()

    for jr in range(_ROWS_PER_W):
        r = base + jr
        in_v = bufs[jr % 2]
        pltpu.make_async_copy(x_hbm.at[r], in_v, sems_in[jr % 2]).wait()
        if jr + 1 < _ROWS_PER_W:
            nxt = bufs[(jr + 1) % 2]
            if jr >= 1:
                pltpu.make_async_copy(
                    nxt, out_hbm.at[r - 1], sems_out[(jr + 1) % 2]).wait()
            pltpu.make_async_copy(x_hbm.at[r + 1], nxt, sems_in[(jr + 1) % 2]).start()

        @plsc.parallel_loop(0, _COLS, 16, unroll=8)
        def p1(o):
            s = in_v[pl.ds(o, 16)]
            ik = s ^ lax.shift_right_logical(lax.shift_right_arithmetic(s, 31), 1)
            key_v[pl.ds(o, 16)] = ik
            b0 = lax.shift_right_arithmetic(ik, 24) + 128
            plsc.addupdate_scatter(histp_ref, [lane_base + b0], ones)

        B0, k1 = _scan_private(histp_ref, jnp.int32(_K))

        @plsc.parallel_loop(0, _COLS, 16, unroll=8)
        def p2(o):
            ik = key_v[pl.ds(o, 16)]
            m = (lax.shift_right_arithmetic(ik, 24) + 128) == B0
            b = jnp.bitwise_and(lax.shift_right_arithmetic(ik, 16), 255)
            plsc.addupdate_scatter(hist_ref, [b], ones, mask=m)

        B1, k2 = _scan_shared(hist_ref, k1)
        t16 = (B0 - 128) * 256 + B1

        @plsc.parallel_loop(0, _COLS, 16, unroll=8)
        def p3(o):
            ik = key_v[pl.ds(o, 16)]
            m = lax.shift_right_arithmetic(ik, 16) == t16
            b = jnp.bitwise_and(lax.shift_right_arithmetic(ik, 8), 255)
            plsc.addupdate_scatter(hist_ref, [b], ones, mask=m)

        B2, k3 = _scan_shared(hist_ref, k2)
        t8 = t16 * 256 + B2

        @plsc.parallel_loop(0, _COLS, 16, unroll=8)
        def p4(o):
            ik = key_v[pl.ds(o, 16)]
            m = lax.shift_right_arithmetic(ik, 8) == t8
            b = jnp.bitwise_and(ik, 255)
            plsc.addupdate_scatter(hist_ref, [b], ones, mask=m)

        B3, _ = _scan_shared(hist_ref, k3)
        thr = t8 * 256 + B3

        @plsc.parallel_loop(0, _COLS, 16, unroll=8)
        def p5(o):
            ik = key_v[pl.ds(o, 16)]
            v = ik ^ lax.shift_right_logical(lax.shift_right_arithmetic(ik, 31), 1)
            in_v[pl.ds(o, 16)] = jnp.where(ik >= thr, v, jnp.int32(0))

        pltpu.make_async_copy(in_v, out_hbm.at[r], sems_out[jr % 2]).start()

    last = _ROWS_PER_W - 1
    pltpu.make_async_copy(
        bufs[(last - 1) % 2], out_hbm.at[base + last - 1],
        sems_out[(last - 1) % 2]).wait()
    pltpu.make_async_copy(
        bufs[last % 2], out_hbm.at[base + last], sems_out[last % 2]).wait()


def kernel(inputs):
    bits = lax.bitcast_convert_type(inputs, jnp.int32)
    out = _sc_ksparse(bits)
    return lax.bitcast_convert_type(out, jnp.float32)


# speedup vs baseline: 1.2085x; 1.2084x over previous
import functools

import jax
import jax.numpy as jnp
from jax import lax
from jax.experimental import pallas as pl
from jax.experimental.pallas import tpu as pltpu
from jax.experimental.pallas import tpu_sc as plsc

_K = 64
_ROWS = 128
_COLS = 32768
_ROWS_PER_W = _ROWS // 32


def _to_key(v):
    s = lax.bitcast_convert_type(v, jnp.int32)
    return s ^ lax.shift_right_logical(lax.shift_right_arithmetic(s, 31), 1)


def _scan_level(hist_ref, k):
    iota = lax.iota(jnp.int32, 16)
    zeros = jnp.zeros(16, jnp.int32)
    ts, gs = [], []
    for i in range(16):
        t = hist_ref[pl.ds(i * 16, 16)]
        ts.append(t)
        gs.append(jnp.sum(t))
        hist_ref[pl.ds(i * 16, 16)] = zeros
    sg = [None] * 17
    sg[16] = jnp.int32(0)
    for i in range(15, -1, -1):
        sg[i] = sg[i + 1] + gs[i]
    G = jnp.int32(0)
    for i in range(16):
        G = jnp.where(sg[i] >= k, jnp.int32(i), G)
    sgn = jnp.int32(0)
    v = ts[0]
    for i in range(16):
        is_g = G == jnp.int32(i)
        sgn = jnp.where(is_g, sg[i + 1], sgn)
        v = jnp.where(is_g, ts[i], v)
    s = lax.rev(plsc.cumsum(lax.rev(v, (0,))), (0,))
    m = (s + sgn) >= k
    bl = jnp.max(jnp.where(m, iota, jnp.int32(-1)))
    hb = jnp.max(jnp.where(iota == bl, v, jnp.int32(0)))
    s_at = jnp.max(jnp.where(iota == bl, s, jnp.int32(0)))
    above = s_at + sgn - hb
    return G * 16 + bl, k - above


_mesh = plsc.VectorSubcoreMesh(core_axis_name="c", subcore_axis_name="s")


@functools.partial(
    pl.kernel,
    out_type=jax.ShapeDtypeStruct((_ROWS, _COLS), jnp.float32),
    mesh=_mesh,
    scratch_types=[
        pltpu.VMEM((_COLS,), jnp.float32),
        pltpu.VMEM((_COLS,), jnp.int32),
        pltpu.VMEM((256,), jnp.int32),
    ],
    compiler_params=pltpu.CompilerParams(needs_layout_passes=False),
)
def _sc_ksparse(x_hbm, out_hbm, row_v, key_v, hist_ref):
    wid = lax.axis_index("s") * 2 + lax.axis_index("c")
    ones = jnp.ones(16, jnp.int32)
    zeros = jnp.zeros(16, jnp.int32)
    for i in range(16):
        hist_ref[pl.ds(i * 16, 16)] = zeros

    def row_body(jr, carry):
        r = wid * _ROWS_PER_W + jr
        pltpu.sync_copy(x_hbm.at[r], row_v)

        @plsc.parallel_loop(0, _COLS, 16, unroll=8)
        def p1(o):
            v = row_v[pl.ds(o, 16)]
            ik = _to_key(v)
            key_v[pl.ds(o, 16)] = ik
            b0 = lax.shift_right_arithmetic(ik, 24) + 128
            plsc.addupdate_scatter(hist_ref, [b0], ones)

        B0, k1 = _scan_level(hist_ref, jnp.int32(_K))

        @plsc.parallel_loop(0, _COLS, 16, unroll=8)
        def p2(o):
            ik = key_v[pl.ds(o, 16)]
            m = (lax.shift_right_arithmetic(ik, 24) + 128) == B0
            b = jnp.bitwise_and(lax.shift_right_arithmetic(ik, 16), 255)
            plsc.addupdate_scatter(hist_ref, [b], ones, mask=m)

        B1, k2 = _scan_level(hist_ref, k1)
        t16 = (B0 - 128) * 256 + B1

        @plsc.parallel_loop(0, _COLS, 16, unroll=8)
        def p3(o):
            ik = key_v[pl.ds(o, 16)]
            m = lax.shift_right_arithmetic(ik, 16) == t16
            b = jnp.bitwise_and(lax.shift_right_arithmetic(ik, 8), 255)
            plsc.addupdate_scatter(hist_ref, [b], ones, mask=m)

        B2, k3 = _scan_level(hist_ref, k2)
        t8 = t16 * 256 + B2

        @plsc.parallel_loop(0, _COLS, 16, unroll=8)
        def p4(o):
            ik = key_v[pl.ds(o, 16)]
            m = lax.shift_right_arithmetic(ik, 8) == t8
            b = jnp.bitwise_and(ik, 255)
            plsc.addupdate_scatter(hist_ref, [b], ones, mask=m)

        B3, _ = _scan_level(hist_ref, k3)
        thr = t8 * 256 + B3

        @plsc.parallel_loop(0, _COLS, 16, unroll=8)
        def p5(o):
            ik = key_v[pl.ds(o, 16)]
            v = row_v[pl.ds(o, 16)]
            row_v[pl.ds(o, 16)] = jnp.where(ik >= thr, v, jnp.float32(0.0))

        pltpu.sync_copy(row_v, out_hbm.at[r])
        return carry

    lax.fori_loop(0, _ROWS_PER_W, row_body, 0)


def kernel(inputs):
    return _sc_ksparse(inputs)
